# baseline (speedup 1.0000x reference)
import functools
import math

import numpy as np
import jax
import jax.numpy as jnp
from jax import lax
from jax.experimental import pallas as pl
from jax.experimental.pallas import tpu as pltpu
from jax.experimental.pallas import tpu_sc as plsc

D_MODEL = 1024
SCALE = math.sqrt(1024.0)
BATCH = 4
SEQ_LEN = 4096

NUM_WORKERS = 32
POS_PER_TILE = SEQ_LEN // NUM_WORKERS
CHUNK = 8
NGROUP = POS_PER_TILE // CHUNK
GROUP_ROWS = BATCH * CHUNK
LANES = 16
VPR = D_MODEL // LANES
NEMB = 3
NPE = 3
PREFETCH = 1


def _make_pe_np():
    position = np.arange(SEQ_LEN, dtype=np.float32)[:, None]
    div_term = np.exp(
        np.arange(0, D_MODEL, 2, dtype=np.float32) * -(math.log(10000.0) / D_MODEL)
    )
    pe = np.zeros((SEQ_LEN, D_MODEL), dtype=np.float32)
    val = position * div_term[None, :]
    pe[:, 0::2] = np.sin(val)
    pe[:, 1::2] = np.cos(val)
    pe2 = pe.reshape(SEQ_LEN, D_MODEL // 32, 2, 16).transpose(0, 1, 3, 2)
    import ml_dtypes
    pe_bf = np.ascontiguousarray(pe2.reshape(SEQ_LEN * D_MODEL)).astype(ml_dtypes.bfloat16)
    return pe_bf.view(np.int32)


_PE = _make_pe_np()

_mesh = plsc.VectorSubcoreMesh(core_axis_name="c", subcore_axis_name="s")


@functools.partial(
    pl.kernel,
    out_type=jax.ShapeDtypeStruct((BATCH, SEQ_LEN, D_MODEL), jnp.float32),
    mesh=_mesh,
    scratch_types=[
        pltpu.VMEM((BATCH, POS_PER_TILE), jnp.int32),
        pltpu.VMEM((NEMB, GROUP_ROWS, D_MODEL), jnp.float32),
        pltpu.VMEM((NPE * CHUNK * (D_MODEL // 2),), jnp.int32),
        pltpu.SemaphoreType.DMA,
        pltpu.SemaphoreType.DMA,
        pltpu.SemaphoreType.DMA,
    ],
)
def _emb_pe(x_hbm, table_hbm, pe_hbm, out_hbm, idx_v, emb_v, pe_v,
            gsem, psem, wsem):
    wid = lax.axis_index("s") * 2 + lax.axis_index("c")
    base = wid * POS_PER_TILE

    idx_copies = [
        pltpu.make_async_copy(
            x_hbm.at[b, pl.ds(base, POS_PER_TILE)], idx_v.at[b], gsem
        )
        for b in range(BATCH)
    ]
    for d in idx_copies:
        d.start()
    for d in idx_copies:
        d.wait()

    def gather_descs(g):
        return [
            pltpu.make_async_copy(
                table_hbm.at[idx_v.at[b, pl.ds(g * CHUNK, CHUNK)]],
                emb_v.at[g % NEMB, pl.ds(b * CHUNK, CHUNK)],
                gsem,
            )
            for b in range(BATCH)
        ]

    def pe_desc(g):
        return pltpu.make_async_copy(
            pe_hbm.at[pl.ds((base + g * CHUNK) * (D_MODEL // 2),
                            CHUNK * (D_MODEL // 2))],
            pe_v.at[pl.ds((g % NPE) * (CHUNK * (D_MODEL // 2)),
                          CHUNK * (D_MODEL // 2))],
            psem,
        )

    def wb_descs(g):
        return [
            pltpu.make_async_copy(
                emb_v.at[g % NEMB, pl.ds(b * CHUNK, CHUNK)],
                out_hbm.at[b, pl.ds(base + g * CHUNK, CHUNK)],
                wsem,
            )
            for b in range(BATCH)
        ]

    for g in range(2):
        pe_desc(g).start()
        for d in gather_descs(g):
            d.start()

    def step(g, _):
        @pl.when(g + PREFETCH - NEMB >= 0)
        def _():
            for d in wb_descs(g + PREFETCH - NEMB):
                d.wait()

        @pl.when(jnp.logical_and(g + PREFETCH >= 2, g + PREFETCH < NGROUP))
        def _():
            for d in gather_descs(g + PREFETCH):
                d.start()
            pe_desc(g + PREFETCH).start()

        pe_desc(g).wait()
        for d in gather_descs(g):
            d.wait()

        pg = g % NEMB

        for slot in range(NEMB):
            @pl.when(pg == slot)
            def _(slot=slot):
                def ew(r, _):
                    for j2 in range(VPR // 2):
                        pev2 = pe_v[
                            pl.ds(slot * (CHUNK * (D_MODEL // 2))
                                  + r * (D_MODEL // 2) + j2 * 16, 16)
                        ]
                        pa = lax.bitcast_convert_type(
                            lax.shift_left(pev2, 16), jnp.float32
                        )
                        pb = lax.bitcast_convert_type(
                            lax.bitwise_and(pev2, jnp.int32(-65536)),
                            jnp.float32,
                        )
                        sl0 = pl.ds(j2 * 32, LANES)
                        sl1 = pl.ds(j2 * 32 + LANES, LANES)
                        for b in range(BATCH):
                            row = b * CHUNK + r
                            emb_v[slot, row, sl0] = (
                                emb_v[slot, row, sl0] * SCALE + pa
                            )
                            emb_v[slot, row, sl1] = (
                                emb_v[slot, row, sl1] * SCALE + pb
                            )
                    return 0

                lax.fori_loop(0, CHUNK, ew, 0)

        for d in wb_descs(g):
            d.start()
        return 0

    lax.fori_loop(0, NGROUP, step, 0)

    for g in range(NGROUP - (NEMB - PREFETCH), NGROUP):
        for d in wb_descs(g):
            d.wait()


_PE_DEV = None


def kernel(x, table):
    global _PE_DEV
    if _PE_DEV is None:
        _PE_DEV = jax.device_put(_PE)
    return _emb_pe(x, table, _PE_DEV)


# speedup vs baseline: 1.0040x; 1.0040x over previous
import functools
import math

import numpy as np
import jax
import jax.numpy as jnp
from jax import lax
from jax.experimental import pallas as pl
from jax.experimental.pallas import tpu as pltpu
from jax.experimental.pallas import tpu_sc as plsc

D_MODEL = 1024
SCALE = math.sqrt(1024.0)
BATCH = 4
SEQ_LEN = 4096

NUM_WORKERS = 32
POS_PER_TILE = SEQ_LEN // NUM_WORKERS
CHUNK = 8
NGROUP = POS_PER_TILE // CHUNK
GROUP_ROWS = BATCH * CHUNK
LANES = 16
VPR = D_MODEL // LANES
NEMB = 3
NPE = 3
PREFETCH = 1


def _make_pe_np():
    position = np.arange(SEQ_LEN, dtype=np.float32)[:, None]
    div_term = np.exp(
        np.arange(0, D_MODEL, 2, dtype=np.float32) * -(math.log(10000.0) / D_MODEL)
    )
    pe = np.zeros((SEQ_LEN, D_MODEL), dtype=np.float32)
    val = position * div_term[None, :]
    pe[:, 0::2] = np.sin(val)
    pe[:, 1::2] = np.cos(val)
    pe2 = pe.reshape(SEQ_LEN, D_MODEL // 32, 2, 16).transpose(0, 1, 3, 2)
    import ml_dtypes
    pe_bf = np.ascontiguousarray(pe2.reshape(SEQ_LEN * D_MODEL)).astype(ml_dtypes.bfloat16)
    return pe_bf.view(np.int32)


_PE = _make_pe_np()

_mesh = plsc.VectorSubcoreMesh(core_axis_name="c", subcore_axis_name="s")


@functools.partial(
    pl.kernel,
    out_type=jax.ShapeDtypeStruct((BATCH, SEQ_LEN, D_MODEL), jnp.float32),
    mesh=_mesh,
    scratch_types=[
        pltpu.VMEM((BATCH, POS_PER_TILE), jnp.int32),
        pltpu.VMEM((NEMB, GROUP_ROWS, D_MODEL), jnp.float32),
        pltpu.VMEM((NPE * CHUNK * (D_MODEL // 2),), jnp.int32),
        pltpu.SemaphoreType.DMA,
        pltpu.SemaphoreType.DMA,
        pltpu.SemaphoreType.DMA,
    ],
)
def _emb_pe(x_hbm, table_hbm, pe_hbm, out_hbm, idx_v, emb_v, pe_v,
            gsem, psem, wsem):
    wid = lax.axis_index("s") * 2 + lax.axis_index("c")
    base = wid * POS_PER_TILE

    idx_copies = [
        pltpu.make_async_copy(
            x_hbm.at[b, pl.ds(base, POS_PER_TILE)], idx_v.at[b], gsem
        )
        for b in range(BATCH)
    ]
    for d in idx_copies:
        d.start()
    def gather_descs(g):
        return [
            pltpu.make_async_copy(
                table_hbm.at[idx_v.at[b, pl.ds(g * CHUNK, CHUNK)]],
                emb_v.at[g % NEMB, pl.ds(b * CHUNK, CHUNK)],
                gsem,
            )
            for b in range(BATCH)
        ]

    def pe_desc(g):
        return pltpu.make_async_copy(
            pe_hbm.at[pl.ds((base + g * CHUNK) * (D_MODEL // 2),
                            CHUNK * (D_MODEL // 2))],
            pe_v.at[pl.ds((g % NPE) * (CHUNK * (D_MODEL // 2)),
                          CHUNK * (D_MODEL // 2))],
            psem,
        )

    def wb_descs(g):
        return [
            pltpu.make_async_copy(
                emb_v.at[g % NEMB, pl.ds(b * CHUNK, CHUNK)],
                out_hbm.at[b, pl.ds(base + g * CHUNK, CHUNK)],
                wsem,
            )
            for b in range(BATCH)
        ]

    for g in range(2):
        pe_desc(g).start()
    for d in idx_copies:
        d.wait()
    for g in range(2):
        for d in gather_descs(g):
            d.start()

    def step(g, _):
        @pl.when(g + PREFETCH - NEMB >= 0)
        def _():
            for d in wb_descs(g + PREFETCH - NEMB):
                d.wait()

        @pl.when(jnp.logical_and(g + PREFETCH >= 2, g + PREFETCH < NGROUP))
        def _():
            for d in gather_descs(g + PREFETCH):
                d.start()
            pe_desc(g + PREFETCH).start()

        pe_desc(g).wait()
        for d in gather_descs(g):
            d.wait()

        pg = g % NEMB

        for slot in range(NEMB):
            @pl.when(pg == slot)
            def _(slot=slot):
                def ew(r, _):
                    for j2 in range(VPR // 2):
                        pev2 = pe_v[
                            pl.ds(slot * (CHUNK * (D_MODEL // 2))
                                  + r * (D_MODEL // 2) + j2 * 16, 16)
                        ]
                        pa = lax.bitcast_convert_type(
                            lax.shift_left(pev2, 16), jnp.float32
                        )
                        pb = lax.bitcast_convert_type(
                            lax.bitwise_and(pev2, jnp.int32(-65536)),
                            jnp.float32,
                        )
                        sl0 = pl.ds(j2 * 32, LANES)
                        sl1 = pl.ds(j2 * 32 + LANES, LANES)
                        for b in range(BATCH):
                            row = b * CHUNK + r
                            emb_v[slot, row, sl0] = (
                                emb_v[slot, row, sl0] * SCALE + pa
                            )
                            emb_v[slot, row, sl1] = (
                                emb_v[slot, row, sl1] * SCALE + pb
                            )
                    return 0

                lax.fori_loop(0, CHUNK, ew, 0)

        for d in wb_descs(g):
            d.start()
        return 0

    lax.fori_loop(0, NGROUP, step, 0)

    for g in range(NGROUP - (NEMB - PREFETCH), NGROUP):
        for d in wb_descs(g):
            d.wait()


_PE_DEV = None


def kernel(x, table):
    global _PE_DEV
    if _PE_DEV is None:
        _PE_DEV = jax.device_put(_PE)
    return _emb_pe(x, table, _PE_DEV)
